# Initial kernel scaffold; baseline (speedup 1.0000x reference)
import jax
import jax.numpy as jnp
from jax.experimental import pallas as pl


def kernel(relative_position_bias_table, relative_position_index):
    raise NotImplementedError("write your pallas kernel here")


# speedup vs baseline: 4.1522x; 4.1522x over previous
import functools

import jax
import jax.numpy as jnp
from jax import lax
from jax.experimental import pallas as pl
from jax.experimental.pallas import tpu as pltpu
from jax.experimental.pallas import tpu_sc as plsc

NUM_REL = 2212
H = 16
N = 577
TOTAL = N * N
NC = 2
NS = 16
NW = NC * NS
L = 16

B = 10496
PAD_TOTAL = NW * B
VPC = B // L
UNROLL = 16
OUTER = VPC // UNROLL


def _sc_bias_gather(table, idx_flat):
    mesh = plsc.VectorSubcoreMesh(core_axis_name="c", subcore_axis_name="s")

    @functools.partial(
        pl.kernel,
        mesh=mesh,
        out_type=jax.ShapeDtypeStruct((H, PAD_TOTAL), jnp.float32),
        compiler_params=pltpu.CompilerParams(needs_layout_passes=False),
        scratch_types=[
            pltpu.VMEM((NUM_REL * H,), jnp.float32),
            pltpu.VMEM((B,), jnp.int32),
            pltpu.VMEM((B,), jnp.float32),
            pltpu.VMEM((B,), jnp.float32),
            pltpu.SemaphoreType.DMA,
            pltpu.SemaphoreType.DMA,
        ],
    )
    def k(table_hbm, idx_hbm, out_hbm, table_v, idx_v, buf0, buf1, sem0, sem1):
        cid = lax.axis_index("c")
        sid = lax.axis_index("s")
        wid = sid * NC + cid
        base = wid * B

        pltpu.sync_copy(table_hbm, table_v)
        pltpu.sync_copy(idx_hbm.at[pl.ds(base, B)], idx_v)

        bufs = (buf0, buf1)
        sems = (sem0, sem1)
        pending = [None, None]

        for h in range(H):
            slot = h % 2
            if pending[slot] is not None:
                pending[slot].wait()
            buf = bufs[slot]

            def body(o, _, buf=buf, h=h):
                for u in range(UNROLL):
                    off = (o * UNROLL + u) * L
                    iv = idx_v[pl.ds(off, L)]
                    vals = plsc.load_gather(table_v, [iv * H + h])
                    buf[pl.ds(off, L)] = vals
                return 0

            lax.fori_loop(0, OUTER, body, 0)

            cp = pltpu.async_copy(buf, out_hbm.at[h, pl.ds(base, B)], sems[slot])
            pending[slot] = cp

        for p in pending:
            if p is not None:
                p.wait()

    return k(table, idx_flat)


def kernel(relative_position_bias_table, relative_position_index):
    table = relative_position_bias_table.astype(jnp.float32).reshape(-1)
    idx = relative_position_index.reshape(-1).astype(jnp.int32)
    idx = jnp.pad(idx, (0, PAD_TOTAL - TOTAL))
    out = _sc_bias_gather(table, idx)
    return out[:, :TOTAL].reshape(H, N, N)


# speedup vs baseline: 5.4555x; 1.3139x over previous
import functools

import jax
import jax.numpy as jnp
from jax import lax
from jax.experimental import pallas as pl
from jax.experimental.pallas import tpu as pltpu
from jax.experimental.pallas import tpu_sc as plsc

NUM_REL = 2212
H = 16
N = 577
TOTAL = N * N
NC = 2
NS = 16
NW = NC * NS
L = 16

B = 10752
PAD_TOTAL = NW * B
CSZ = 2688
NCHUNK = B // CSZ
VPC = CSZ // L


def _sc_bias_gather(table, idx_flat):
    mesh = plsc.VectorSubcoreMesh(core_axis_name="c", subcore_axis_name="s")

    @functools.partial(
        pl.kernel,
        mesh=mesh,
        out_type=jax.ShapeDtypeStruct((H, PAD_TOTAL), jnp.float32),
        compiler_params=pltpu.CompilerParams(needs_layout_passes=False),
        scratch_types=[
            pltpu.VMEM((NUM_REL * H,), jnp.float32),
            pltpu.VMEM((CSZ,), jnp.int32),
            pltpu.VMEM((H, CSZ), jnp.float32),
            pltpu.VMEM((H, CSZ), jnp.float32),
            pltpu.SemaphoreType.DMA,
            pltpu.SemaphoreType.DMA,
        ],
    )
    def k(table_hbm, idx_hbm, out_hbm, table_v, idx_v, buf0, buf1, sem0, sem1):
        cid = lax.axis_index("c")
        sid = lax.axis_index("s")
        wid = sid * NC + cid
        base = wid * B

        pltpu.sync_copy(table_hbm, table_v)

        bufs = (buf0, buf1)
        sems = (sem0, sem1)
        pending = [None, None]

        for c in range(NCHUNK):
            slot = c % 2
            if pending[slot] is not None:
                pending[slot].wait()
            buf = bufs[slot]
            pltpu.sync_copy(idx_hbm.at[pl.ds(base + c * CSZ, CSZ)], idx_v)

            @plsc.parallel_loop(0, VPC, 1, unroll=4)
            def _(v, buf=buf):
                off = v * L
                iv = idx_v[pl.ds(off, L)]
                iv16 = iv * H
                for h in range(H):
                    vals = plsc.load_gather(table_v, [iv16 + h])
                    buf[h, pl.ds(off, L)] = vals

            pending[slot] = pltpu.async_copy(
                buf, out_hbm.at[:, pl.ds(base + c * CSZ, CSZ)], sems[slot]
            )

        for p in pending:
            if p is not None:
                p.wait()

    return k(table, idx_flat)


def kernel(relative_position_bias_table, relative_position_index):
    table = relative_position_bias_table.astype(jnp.float32).reshape(-1)
    idx = relative_position_index.reshape(-1).astype(jnp.int32)
    idx = jnp.pad(idx, (0, PAD_TOTAL - TOTAL))
    out = _sc_bias_gather(table, idx)
    return out[:, :TOTAL].reshape(H, N, N)


# speedup vs baseline: 6.7000x; 1.2281x over previous
import functools

import jax
import jax.numpy as jnp
from jax import lax
from jax.experimental import pallas as pl
from jax.experimental.pallas import tpu as pltpu
from jax.experimental.pallas import tpu_sc as plsc

NUM_REL = 2212
H = 16
N = 577
TOTAL = N * N
NC = 2
NS = 16
NW = NC * NS
L = 16

B = 10752
PAD_TOTAL = NW * B
CSZ = 2688
NCHUNK = B // CSZ
VPC = CSZ // L


def _sc_bias_gather(table, idx_flat):
    mesh = plsc.VectorSubcoreMesh(core_axis_name="c", subcore_axis_name="s")

    @functools.partial(
        pl.kernel,
        mesh=mesh,
        out_type=jax.ShapeDtypeStruct((H, PAD_TOTAL), jnp.float32),
        compiler_params=pltpu.CompilerParams(needs_layout_passes=False),
        scratch_types=[
            pltpu.VMEM((NUM_REL * H,), jnp.float32),
            pltpu.VMEM((CSZ,), jnp.int32),
            pltpu.VMEM((H, CSZ), jnp.float32),
            pltpu.VMEM((H, CSZ), jnp.float32),
            pltpu.SemaphoreType.DMA,
            pltpu.SemaphoreType.DMA,
        ],
    )
    def k(table_hbm, idx_hbm, out_hbm, table_v, idx_v, buf0, buf1, sem0, sem1):
        cid = lax.axis_index("c")
        sid = lax.axis_index("s")
        wid = sid * NC + cid
        base = wid * B

        pltpu.sync_copy(table_hbm, table_v)

        bufs = (buf0, buf1)
        sems = (sem0, sem1)
        pending = [None, None]

        for c in range(NCHUNK):
            slot = c % 2
            if pending[slot] is not None:
                pending[slot].wait()
            buf = bufs[slot]
            pltpu.sync_copy(idx_hbm.at[pl.ds(base + c * CSZ, CSZ)], idx_v)

            @plsc.parallel_loop(0, VPC, 1, unroll=4)
            def _(v, buf=buf):
                off = v * L
                iv = idx_v[pl.ds(off, L)]
                for h in range(H):
                    vals = plsc.load_gather(table_v, [iv + h * NUM_REL])
                    buf[h, pl.ds(off, L)] = vals

            pending[slot] = pltpu.async_copy(
                buf, out_hbm.at[:, pl.ds(base + c * CSZ, CSZ)], sems[slot]
            )

        for p in pending:
            if p is not None:
                p.wait()

    return k(table, idx_flat)


def kernel(relative_position_bias_table, relative_position_index):
    table = relative_position_bias_table.astype(jnp.float32).T.reshape(-1)
    idx = relative_position_index.reshape(-1).astype(jnp.int32)
    idx = jnp.pad(idx, (0, PAD_TOTAL - TOTAL))
    out = _sc_bias_gather(table, idx)
    return out[:, :TOTAL].reshape(H, N, N)
